# baseline (speedup 1.0000x reference)
import functools

import jax
import jax.numpy as jnp
from jax import lax
from jax.experimental import pallas as pl
from jax.experimental.pallas import tpu as pltpu
from jax.experimental.pallas import tpu_sc as plsc

_OUTPUT_DIM = 64
_NB_ACTIONS = 26
_BATCH = 16384

_NC = 2
_NS = 16
_NW = _NC * _NS
_L = 16
_BPW = _BATCH // _NW
_CH = 128
_NCH = _BPW // _CH

_mesh = plsc.VectorSubcoreMesh(core_axis_name="c", subcore_axis_name="s")


@functools.partial(
    pl.kernel,
    mesh=_mesh,
    out_type=jax.ShapeDtypeStruct((_BATCH, _OUTPUT_DIM), jnp.float32),
    scratch_types=[
        pltpu.VMEM((_BPW,), jnp.int32),
        pltpu.VMEM((_NCH, _CH), jnp.int32),
        pltpu.VMEM((2, _CH, 128), jnp.float32),
        pltpu.VMEM((_BPW, _OUTPUT_DIM), jnp.float32),
        pltpu.SemaphoreType.DMA,
        pltpu.SemaphoreType.DMA,
    ],
)
def _gather_rows(
    table_hbm, idx_hbm, out_hbm, rawidx_v, rowid_v, rows_pp, sel_v, gsem, wsem
):
    wid = lax.axis_index("s") * _NC + lax.axis_index("c")
    base = wid * _BPW

    pltpu.sync_copy(idx_hbm.at[pl.ds(base, _BPW)], rawidx_v)

    lane = lax.iota(jnp.int32, 16)
    for j in range(_NCH):
        for i in range(_CH // _L):
            k = j * (_CH // _L) + i
            act = rawidx_v[pl.ds(k * _L, _L)]
            b = base + k * _L + lane
            rowid_v[j, pl.ds(i * _L, _L)] = (
                ((b >> 3) * 13 + (act >> 1)) * 8 + (b & 7)
            )

    def _fire(j):
        return pltpu.async_copy(
            table_hbm.at[rowid_v.at[j]],
            rows_pp.at[j % 2],
            gsem,
        )

    gather = _fire(0)
    writes = []
    for j in range(_NCH):
        nxt = _fire(j + 1) if j + 1 < _NCH else None
        gather.wait()

        @pl.loop(0, _CH // _L)
        def _select_group(g, _j=j):
            r0 = _j * _CH + g * _L
            act16 = rawidx_v[pl.ds(r0, _L)]
            for i in range(_L):
                half = (act16[i] & 1) * _OUTPUT_DIM
                rr = g * _L + i
                for j0 in range(0, _OUTPUT_DIM, _L):
                    sel_v[_j * _CH + rr, pl.ds(j0, _L)] = rows_pp[
                        _j % 2, rr, pl.ds(half + j0, _L)
                    ]

        writes.append(
            pltpu.async_copy(
                sel_v.at[pl.ds(j * _CH, _CH)],
                out_hbm.at[pl.ds(base + j * _CH, _CH)],
                wsem,
            )
        )
        gather = nxt
    for w in writes:
        w.wait()


def kernel(full_output, indices):
    table = (
        full_output.reshape(_BATCH // 8, 8, _NB_ACTIONS // 2, 128)
        .swapaxes(1, 2)
        .reshape(_BATCH * _NB_ACTIONS // 2, 128)
    )
    idx = indices.reshape(_BATCH).astype(jnp.int32)
    return _gather_rows(table, idx)


# speedup vs baseline: 1.2764x; 1.2764x over previous
import functools

import jax
import jax.numpy as jnp
from jax import lax
from jax.experimental import pallas as pl
from jax.experimental.pallas import tpu as pltpu
from jax.experimental.pallas import tpu_sc as plsc

_OUTPUT_DIM = 64
_NB_ACTIONS = 26
_BATCH = 16384

_NC = 2
_NS = 16
_NW = _NC * _NS
_L = 16
_BPW = _BATCH // _NW
_CH = 128
_NCH = _BPW // _CH

_mesh = plsc.VectorSubcoreMesh(core_axis_name="c", subcore_axis_name="s")


@functools.partial(
    pl.kernel,
    mesh=_mesh,
    out_type=jax.ShapeDtypeStruct((_BATCH, 2 * _OUTPUT_DIM), jnp.float32),
    scratch_types=[
        pltpu.VMEM((_BPW,), jnp.int32),
        pltpu.VMEM((_NCH, _CH), jnp.int32),
        pltpu.VMEM((_BPW, _OUTPUT_DIM), jnp.float32),
        pltpu.SemaphoreType.DMA,
        pltpu.SemaphoreType.DMA,
    ],
    compiler_params=pltpu.CompilerParams(use_tc_tiling_on_sc=False),
)
def _gather_rows(table_hbm, idx_hbm, out_hbm, rawidx_v, rowid_v, rows_v, gsem, wsem):
    wid = lax.axis_index("s") * _NC + lax.axis_index("c")
    base = wid * _BPW

    pltpu.sync_copy(idx_hbm.at[pl.ds(base, _BPW)], rawidx_v)

    lane = lax.iota(jnp.int32, 16)

    @pl.loop(0, _BPW // _L)
    def _rowids(k):
        act = rawidx_v[pl.ds(k * _L, _L)]
        b = base + k * _L + lane
        rowid_v[k // (_CH // _L), pl.ds((k % (_CH // _L)) * _L, _L)] = (
            ((b >> 3) * 13 + (act >> 1)) * 16 + ((b & 7) << 1) + (act & 1)
        )

    gathers = []
    for j in range(_NCH):
        gathers.append(
            pltpu.async_copy(
                table_hbm.at[rowid_v.at[j]],
                rows_v.at[pl.ds(j * _CH, _CH)],
                gsem,
            )
        )

    writes = []
    for j in range(_NCH):
        gathers[j].wait()
        writes.append(
            pltpu.async_copy(
                rows_v.at[pl.ds(j * _CH, _CH)],
                out_hbm.at[pl.ds(base + j * _CH, _CH), pl.ds(0, _OUTPUT_DIM)],
                wsem,
            )
        )
    for w in writes:
        w.wait()


def kernel(full_output, indices):
    table = (
        full_output.reshape(_BATCH // 8, 8, _NB_ACTIONS // 2, 128)
        .swapaxes(1, 2)
        .reshape(_BATCH * _NB_ACTIONS, _OUTPUT_DIM)
    )
    idx = indices.reshape(_BATCH).astype(jnp.int32)
    padded = _gather_rows(table, idx)
    return padded[:, :_OUTPUT_DIM]
